# baseline (speedup 1.0000x reference)
import functools

import jax
import jax.numpy as jnp
from jax import lax
from jax.experimental import pallas as pl
from jax.experimental.pallas import tpu as pltpu
from jax.experimental.pallas import tpu_sc as plsc

VOCAB = 100000
D = 64
T = 200
B = 4096
NC = 2
NS = 16
NW = NC * NS
BB = B // NW
LANES = 16
NSTEP = T // 2


def _emb_body(ids_hbm, tok_hbm, pos_hbm, out_hbm,
              idx0, idx1, rows0, rows1, tr0, tr1, pos_v,
              gsem0, gsem1, wsem0, wsem1):
    wid = lax.axis_index("s") * NC + lax.axis_index("c")
    bcol = wid * BB
    pltpu.sync_copy(pos_hbm, pos_v)

    idx = (idx0, idx1)
    rows = (rows0, rows1)
    tr = (tr0, tr1)
    gsem = (gsem0, gsem1)
    wsem = (wsem0, wsem1)

    iota = lax.iota(jnp.int32, LANES)
    row_idx = [iota + LANES * i for i in range(BB // LANES)]

    def transpose_add(rbuf, tbuf, t):
        def d_body(d, c):
            col = jnp.full((LANES,), d, jnp.int32)
            pv = plsc.load_gather(pos_v, [jnp.full((LANES,), d * T + t, jnp.int32)])
            for i in range(BB // LANES):
                v = plsc.load_gather(rbuf, [row_idx[i], col])
                tbuf[d, pl.ds(LANES * i, LANES)] = v + pv
            return c
        lax.fori_loop(0, D, d_body, 0)

    pltpu.sync_copy(ids_hbm.at[0, pl.ds(bcol, BB)], idx0)
    pltpu.async_copy(tok_hbm.at[idx0], rows0, gsem0)

    def step_body(s, carry):
        for b in range(2):
            t = 2 * s + b
            nb = 1 - b
            if b == 0:
                pltpu.sync_copy(ids_hbm.at[t + 1, pl.ds(bcol, BB)], idx[nb])

                @pl.when(s > 0)
                def _wait_prev_write():
                    pltpu.make_async_copy(
                        tr[nb], out_hbm.at[t - 1, :, pl.ds(bcol, BB)], wsem[nb]
                    ).wait()

                pltpu.async_copy(tok_hbm.at[idx[nb]], rows[nb], gsem[nb])
            else:
                @pl.when(s < NSTEP - 1)
                def _stage_next():
                    pltpu.sync_copy(ids_hbm.at[t + 1, pl.ds(bcol, BB)], idx[nb])
                    pltpu.make_async_copy(
                        tr[nb], out_hbm.at[t - 1, :, pl.ds(bcol, BB)], wsem[nb]
                    ).wait()
                    pltpu.async_copy(tok_hbm.at[idx[nb]], rows[nb], gsem[nb])

            pltpu.make_async_copy(tok_hbm.at[idx[b]], rows[b], gsem[b]).wait()
            transpose_add(rows[b], tr[b], t)
            pltpu.async_copy(tr[b], out_hbm.at[t, :, pl.ds(bcol, BB)], wsem[b])
        return carry

    lax.fori_loop(0, NSTEP, step_body, 0)

    pltpu.make_async_copy(
        tr0, out_hbm.at[T - 2, :, pl.ds(bcol, BB)], wsem0
    ).wait()
    pltpu.make_async_copy(
        tr1, out_hbm.at[T - 1, :, pl.ds(bcol, BB)], wsem1
    ).wait()


_emb_kernel = functools.partial(
    pl.kernel,
    out_type=jax.ShapeDtypeStruct((T, D, B), jnp.float32),
    mesh=plsc.VectorSubcoreMesh(core_axis_name="c", subcore_axis_name="s"),
    scratch_types=[
        pltpu.VMEM((BB,), jnp.int32),
        pltpu.VMEM((BB,), jnp.int32),
        pltpu.VMEM((BB, D), jnp.float32),
        pltpu.VMEM((BB, D), jnp.float32),
        pltpu.VMEM((D, BB), jnp.float32),
        pltpu.VMEM((D, BB), jnp.float32),
        pltpu.VMEM((D * T,), jnp.float32),
        pltpu.SemaphoreType.DMA,
        pltpu.SemaphoreType.DMA,
        pltpu.SemaphoreType.DMA,
        pltpu.SemaphoreType.DMA,
    ],
    compiler_params=pltpu.CompilerParams(
        use_tc_tiling_on_sc=False, needs_layout_passes=False),
)(_emb_body)


def kernel(input_ids, tok_table, pos_table):
    batch, block = input_ids.shape
    ids_t = input_ids.T.astype(jnp.int32)
    pos_t = pos_table.T.reshape(-1)
    out_t = _emb_kernel(ids_t, tok_table, pos_t)
    return jnp.transpose(out_t, (2, 0, 1))
